# baseline (speedup 1.0000x reference)
import functools

import jax
import jax.numpy as jnp
from jax import lax
from jax.experimental import pallas as pl
from jax.experimental.pallas import tpu as pltpu
from jax.experimental.pallas import tpu_sc as plsc

N = 10000
D = 128
E = 320000

NC = 2
NS = 16
NW = NC * NS
DH = D // NC

NP = 10240
RPT = NP // NS
EP = 327680
EPW = EP // NW
EPS = EP // NS
CHUNK = 128
NCH = EPS // CHUNK

DH3 = 32

RB = 1024
GRID = NP // RB

NBUF = 5


def _mesh():
    return plsc.VectorSubcoreMesh(core_axis_name="c", subcore_axis_name="s")


_SC_PARAMS = pltpu.CompilerParams(needs_layout_passes=False)
_SC_PARAMS_LINEAR = pltpu.CompilerParams(
    needs_layout_passes=False, use_tc_tiling_on_sc=False)


def _deg_body(dst_hbm, out_hbm, dst_v, hist_v):
    c = lax.axis_index("c")
    s = lax.axis_index("s")
    wid = s * NC + c
    pltpu.sync_copy(dst_hbm.at[pl.ds(wid * EPW, EPW)], dst_v)

    zeros16 = jnp.zeros((16,), jnp.float32)

    def zero_body(i, carry):
        hist_v[pl.ds(i * 16, 16)] = zeros16
        return carry

    lax.fori_loop(0, NP // 16, zero_body, 0)

    ones16 = jnp.ones((16,), jnp.float32)

    def body(i, carry):
        idx = dst_v[pl.ds(i * 16, 16)]
        plsc.addupdate_scatter(hist_v, [idx], ones16)
        return carry

    lax.fori_loop(0, EPW // 16, body, 0)
    pltpu.sync_copy(hist_v, out_hbm.at[wid])


def _degree_histogram(dst_pad):
    k = pl.kernel(
        _deg_body,
        out_type=jax.ShapeDtypeStruct((NW, NP), jnp.float32),
        mesh=_mesh(),
        compiler_params=_SC_PARAMS,
        scratch_types=[
            pltpu.VMEM((EPW,), jnp.int32),
            pltpu.VMEM((NP,), jnp.float32),
        ],
    )
    return k(dst_pad)


def _spmm_body(h_hbm, idx_hbm, out_hbm,
               y_sh, h_sh, ibufs, rows, isems, gsems, ssems, psem):
    c = lax.axis_index("c")
    s = lax.axis_index("s")
    ibase = s * NCH

    def issue_i(ci, b):
        pltpu.async_copy(idx_hbm.at[ibase + ci], ibufs[b], isems[b])

    def wait_i(ci, b):
        pltpu.make_async_copy(idx_hbm.at[ibase + ci], ibufs[b],
                              isems[b]).wait()

    def issue_g(ci, b):
        pltpu.async_copy(h_sh.at[ibufs[b].at[0]], rows[b], gsems[b])

    def wait_g(ci, b):
        pltpu.make_async_copy(h_sh.at[ibufs[b].at[0]], rows[b],
                              gsems[b]).wait()

    def issue_s(ci, b):
        pltpu.async_copy(rows[b], y_sh.at[ibufs[b].at[1]], ssems[b],
                         add=True)

    def wait_s(ci, b):
        pltpu.make_async_copy(rows[b], y_sh.at[ibufs[b].at[1]],
                              ssems[b]).wait()

    def visit(ci, b, f_wait_s, f_issue_i, f_issue_g):
        bs = (b + 2) % NBUF
        bg = (b + 1) % NBUF
        if f_wait_s:
            wait_s(ci - 3, bs)
        if f_issue_i:
            issue_i(ci + 2, bs)
        if f_issue_g:
            wait_i(ci + 1, bg)
            issue_g(ci + 1, bg)
        wait_g(ci, b)
        issue_s(ci, b)

    pltpu.async_copy(h_hbm.at[c, pl.ds(s * RPT, RPT)],
                     h_sh.at[pl.ds(s * RPT, RPT)], psem)
    issue_i(0, 0)
    issue_i(1, 1)
    pltpu.make_async_copy(h_hbm.at[c, pl.ds(s * RPT, RPT)],
                          h_sh.at[pl.ds(s * RPT, RPT)], psem).wait()
    pltpu.sync_copy(h_sh.at[pl.ds(s * RPT, RPT)],
                    y_sh.at[pl.ds(s * RPT, RPT)])
    plsc.subcore_barrier()
    wait_i(0, 0)
    issue_g(0, 0)

    for k in range(NBUF):
        visit(k, k, k >= 3, True, True)

    def round_body(o, carry):
        c0 = o * NBUF
        for k in range(NBUF):
            visit(c0 + k, k, True, True, True)
        return carry

    lax.fori_loop(1, NCH // NBUF - 1, round_body, 0)

    cl = NCH - NBUF
    for k in range(NBUF):
        visit(cl + k, k, True, cl + k + 2 < NCH, cl + k + 1 < NCH)
    for ci in range(NCH - 3, NCH):
        wait_s(ci, ci % NBUF)

    plsc.subcore_barrier()
    pltpu.sync_copy(y_sh.at[pl.ds(s * RPT, RPT)],
                    out_hbm.at[c, pl.ds(s * RPT, RPT)])


def _scatter_rows(h_split, idx2d, dh):
    k = pl.kernel(
        _spmm_body,
        out_type=jax.ShapeDtypeStruct((NC, NP, dh), jnp.float32),
        mesh=_mesh(),
        compiler_params=_SC_PARAMS_LINEAR,
        scratch_types=[
            pltpu.VMEM_SHARED((NP, dh), jnp.float32),
            pltpu.VMEM_SHARED((NP, dh), jnp.float32),
            [pltpu.VMEM((2, CHUNK), jnp.int32) for _ in range(NBUF)],
            [pltpu.VMEM((CHUNK, dh), jnp.float32) for _ in range(NBUF)],
            [pltpu.SemaphoreType.DMA for _ in range(NBUF)],
            [pltpu.SemaphoreType.DMA for _ in range(NBUF)],
            [pltpu.SemaphoreType.DMA for _ in range(NBUF)],
            pltpu.SemaphoreType.DMA,
        ],
    )
    return k(h_split, idx2d)


def _dinv_of(deg_ref):
    deg = jnp.sum(deg_ref[...], axis=0) + 1.0
    return lax.rsqrt(deg)


def _split(h, dh):
    return jnp.stack([h[:, :dh], h[:, dh:2 * dh]])


def _tc_first_body(x_ref, w_ref, deg_ref, o_ref):
    dinv = _dinv_of(deg_ref)
    h = jnp.dot(x_ref[...], w_ref[...], preferred_element_type=jnp.float32)
    o_ref[...] = _split(h * dinv[:, None], DH)


def _tc_mid_body(dh_out, s_ref, deg_ref, w_ref, cb_ref, o_ref):
    dinv = _dinv_of(deg_ref)
    sv = s_ref[...]
    z = jnp.concatenate([sv[0], sv[1]], axis=1)
    z = z * dinv[:, None]
    cb = cb_ref[...]
    xn = jnp.maximum(z * cb[0][None, :] + cb[1][None, :], 0.0)
    h = jnp.dot(xn, w_ref[...], preferred_element_type=jnp.float32)
    o_ref[...] = _split(h * dinv[:, None], dh_out)


def _tc_out_body(s_ref, deg_ref, b_ref, o_ref):
    dinv = _dinv_of(deg_ref)
    sv = s_ref[...]
    z = jnp.concatenate([sv[0], sv[1]], axis=1)
    o_ref[...] = z * dinv[:, None] + b_ref[...]


_ROWS = pl.BlockSpec((RB, D), lambda i: (i, 0))
_WMAT = pl.BlockSpec((D, D), lambda i: (0, 0))
_DEGP = pl.BlockSpec((NW, RB), lambda i: (0, i))


def _splt(dh):
    return pl.BlockSpec((NC, RB, dh), lambda i: (0, i, 0))


def _outs(dh):
    return jax.ShapeDtypeStruct((NC, NP, dh), jnp.float32)


def _tc_first(x_pad, w, deg_parts):
    return pl.pallas_call(
        _tc_first_body, grid=(GRID,),
        in_specs=[_ROWS, _WMAT, _DEGP],
        out_specs=_splt(DH), out_shape=_outs(DH),
    )(x_pad, w, deg_parts)


def _tc_mid(s, deg_parts, w, cb, dh_out):
    return pl.pallas_call(
        functools.partial(_tc_mid_body, dh_out), grid=(GRID,),
        in_specs=[_splt(DH), _DEGP,
                  pl.BlockSpec((D, 2 * dh_out), lambda i: (0, 0)),
                  pl.BlockSpec((2, D), lambda i: (0, 0))],
        out_specs=_splt(dh_out), out_shape=_outs(dh_out),
    )(s, deg_parts, w, cb)


def _tc_out(s, deg_parts, b):
    return pl.pallas_call(
        _tc_out_body, grid=(GRID,),
        in_specs=[_splt(DH3), _DEGP,
                  pl.BlockSpec((1, 2 * DH3), lambda i: (0, 0))],
        out_specs=pl.BlockSpec((RB, 2 * DH3), lambda i: (i, 0)),
        out_shape=jax.ShapeDtypeStruct((NP, 2 * DH3), jnp.float32),
    )(s, deg_parts, b)


@jax.jit
def kernel(x, edge_index, W1, b1, W2, b2, W3, b3, g1, beta1, g2, beta2):
    src = edge_index[0]
    dst = edge_index[1]
    pad = EP - E
    src_pad = jnp.concatenate([src, jnp.zeros((pad,), jnp.int32)])
    dst_pad = jnp.concatenate([dst, jnp.full((pad,), N, jnp.int32)])
    src2d = src_pad.reshape(EP // CHUNK, CHUNK)
    dst2d = dst_pad.reshape(EP // CHUNK, CHUNK)
    idx2d = jnp.stack([src2d, dst2d], axis=1)
    x_pad = jnp.pad(x, ((0, NP - N), (0, 0)))

    bn = 1.0 / jnp.sqrt(jnp.float32(1.0 + 1e-5))
    c1 = g1 * bn
    cb1 = jnp.stack([c1, c1 * b1 + beta1])
    c2 = g2 * bn
    cb2 = jnp.stack([c2, c2 * b2 + beta2])
    w3_pad = jnp.pad(W3, ((0, 0), (0, 2 * DH3 - W3.shape[1])))
    b3_pad = jnp.pad(b3, (0, 2 * DH3 - b3.shape[0]))[None, :]

    deg_parts = _degree_histogram(dst_pad)

    h1 = _tc_first(x_pad, W1, deg_parts)
    s1 = _scatter_rows(h1, idx2d, DH)
    h2 = _tc_mid(s1, deg_parts, W2, cb1, DH)
    s2 = _scatter_rows(h2, idx2d, DH)
    h3 = _tc_mid(s2, deg_parts, w3_pad, cb2, DH3)
    s3 = _scatter_rows(h3, idx2d, DH3)
    out = _tc_out(s3, deg_parts, b3_pad)
    return out[:N, :W3.shape[1]]


# speedup vs baseline: 1.8293x; 1.8293x over previous
import functools

import jax
import jax.numpy as jnp
from jax import lax
from jax.experimental import pallas as pl
from jax.experimental.pallas import tpu as pltpu
from jax.experimental.pallas import tpu_sc as plsc

N = 10000
D = 128
E = 320000

NC = 2
NS = 16
NW = NC * NS
DH = D // NC

NP = 10240
RPT = NP // NS
EP = 327680
EPW = EP // NW
EPS = EP // NS
CHUNK = 128
NCH = EPS // CHUNK

DH3 = 32

RB = 1024
GRID = NP // RB

NBUF = 5


def _mesh():
    return plsc.VectorSubcoreMesh(core_axis_name="c", subcore_axis_name="s")


_SC_PARAMS = pltpu.CompilerParams(needs_layout_passes=False)
_SC_PARAMS_LINEAR = pltpu.CompilerParams(
    needs_layout_passes=False, use_tc_tiling_on_sc=False)


def _deg_body(dst_hbm, out_hbm, dst_v, hist_v):
    c = lax.axis_index("c")
    s = lax.axis_index("s")
    wid = s * NC + c
    pltpu.sync_copy(dst_hbm.at[pl.ds(wid * EPW, EPW)], dst_v)

    zeros16 = jnp.zeros((16,), jnp.float32)

    def zero_body(i, carry):
        hist_v[pl.ds(i * 16, 16)] = zeros16
        return carry

    lax.fori_loop(0, NP // 16, zero_body, 0)

    ones16 = jnp.ones((16,), jnp.float32)

    def body(i, carry):
        idx = dst_v[pl.ds(i * 16, 16)]
        plsc.addupdate_scatter(hist_v, [idx], ones16)
        return carry

    lax.fori_loop(0, EPW // 16, body, 0)
    pltpu.sync_copy(hist_v, out_hbm.at[wid])


def _degree_histogram(dst_pad):
    k = pl.kernel(
        _deg_body,
        out_type=jax.ShapeDtypeStruct((NW, NP), jnp.float32),
        mesh=_mesh(),
        compiler_params=_SC_PARAMS,
        scratch_types=[
            pltpu.VMEM((EPW,), jnp.int32),
            pltpu.VMEM((NP,), jnp.float32),
        ],
    )
    return k(dst_pad)


def _spmm_body(h_hbm, idx_hbm, out_hbm,
               y_sh, h_sh, ibufs, rows, isems, gsems, ssems, psem, qsem):
    c = lax.axis_index("c")
    s = lax.axis_index("s")
    ibase = s * NCH

    def issue_i(ci, b):
        pltpu.async_copy(idx_hbm.at[ibase + ci], ibufs[b], isems[b])

    def wait_i(ci, b):
        pltpu.make_async_copy(idx_hbm.at[ibase + ci], ibufs[b],
                              isems[b]).wait()

    def issue_g(ci, b):
        pltpu.async_copy(h_sh.at[ibufs[b].at[0]], rows[b], gsems[b])

    def wait_g(ci, b):
        pltpu.make_async_copy(h_sh.at[ibufs[b].at[0]], rows[b],
                              gsems[b]).wait()

    def issue_s(ci, b):
        pltpu.async_copy(rows[b], y_sh.at[ibufs[b].at[1]], ssems[b],
                         add=True)

    def wait_s(ci, b):
        pltpu.make_async_copy(rows[b], y_sh.at[ibufs[b].at[1]],
                              ssems[b]).wait()

    def visit(ci, b, f_wait_s, f_issue_i, f_issue_g):
        bs = (b + 2) % NBUF
        bg = (b + 1) % NBUF
        if f_wait_s:
            wait_s(ci - 3, bs)
        if f_issue_i:
            issue_i(ci + 2, bs)
        if f_issue_g:
            wait_i(ci + 1, bg)
            issue_g(ci + 1, bg)
        wait_g(ci, b)
        issue_s(ci, b)

    pltpu.async_copy(h_hbm.at[c, pl.ds(s * RPT, RPT)],
                     h_sh.at[pl.ds(s * RPT, RPT)], psem)
    pltpu.async_copy(h_hbm.at[c, pl.ds(s * RPT, RPT)],
                     y_sh.at[pl.ds(s * RPT, RPT)], qsem)
    issue_i(0, 0)
    issue_i(1, 1)
    pltpu.make_async_copy(h_hbm.at[c, pl.ds(s * RPT, RPT)],
                          h_sh.at[pl.ds(s * RPT, RPT)], psem).wait()
    pltpu.make_async_copy(h_hbm.at[c, pl.ds(s * RPT, RPT)],
                          y_sh.at[pl.ds(s * RPT, RPT)], qsem).wait()
    plsc.subcore_barrier()
    wait_i(0, 0)
    issue_g(0, 0)

    for k in range(NBUF):
        visit(k, k, k >= 3, True, True)

    def round_body(o, carry):
        c0 = o * NBUF
        for k in range(NBUF):
            visit(c0 + k, k, True, True, True)
        return carry

    lax.fori_loop(1, NCH // NBUF - 1, round_body, 0)

    cl = NCH - NBUF
    for k in range(NBUF):
        visit(cl + k, k, True, cl + k + 2 < NCH, cl + k + 1 < NCH)
    for ci in range(NCH - 3, NCH):
        wait_s(ci, ci % NBUF)

    plsc.subcore_barrier()
    pltpu.sync_copy(y_sh.at[pl.ds(s * RPT, RPT)],
                    out_hbm.at[c, pl.ds(s * RPT, RPT)])


def _scatter_rows(h_split, idx2d, dh):
    k = pl.kernel(
        _spmm_body,
        out_type=jax.ShapeDtypeStruct((NC, NP, dh), jnp.float32),
        mesh=_mesh(),
        compiler_params=_SC_PARAMS_LINEAR,
        scratch_types=[
            pltpu.VMEM_SHARED((NP, dh), jnp.float32),
            pltpu.VMEM_SHARED((NP, dh), jnp.float32),
            [pltpu.VMEM((2, CHUNK), jnp.int32) for _ in range(NBUF)],
            [pltpu.VMEM((CHUNK, dh), jnp.float32) for _ in range(NBUF)],
            [pltpu.SemaphoreType.DMA for _ in range(NBUF)],
            [pltpu.SemaphoreType.DMA for _ in range(NBUF)],
            [pltpu.SemaphoreType.DMA for _ in range(NBUF)],
            pltpu.SemaphoreType.DMA,
            pltpu.SemaphoreType.DMA,
        ],
    )
    return k(h_split, idx2d)


def _dinv_of(deg_ref):
    deg = jnp.sum(deg_ref[...], axis=0) + 1.0
    return lax.rsqrt(deg)


def _split(h, dh):
    return jnp.stack([h[:, :dh], h[:, dh:2 * dh]])


def _tc_first_body(x_ref, w_ref, deg_ref, o_ref):
    dinv = _dinv_of(deg_ref)
    h = jnp.dot(x_ref[...], w_ref[...], preferred_element_type=jnp.float32)
    o_ref[...] = _split(h * dinv[:, None], DH)


def _tc_mid_body(dh_out, s_ref, deg_ref, w_ref, cb_ref, o_ref):
    dinv = _dinv_of(deg_ref)
    sv = s_ref[...]
    z = jnp.concatenate([sv[0], sv[1]], axis=1)
    z = z * dinv[:, None]
    cb = cb_ref[...]
    xn = jnp.maximum(z * cb[0][None, :] + cb[1][None, :], 0.0)
    h = jnp.dot(xn, w_ref[...], preferred_element_type=jnp.float32)
    o_ref[...] = _split(h * dinv[:, None], dh_out)


def _tc_out_body(s_ref, deg_ref, b_ref, o_ref):
    dinv = _dinv_of(deg_ref)
    sv = s_ref[...]
    z = jnp.concatenate([sv[0], sv[1]], axis=1)
    o_ref[...] = z * dinv[:, None] + b_ref[...]


_ROWS = pl.BlockSpec((RB, D), lambda i: (i, 0))
_WMAT = pl.BlockSpec((D, D), lambda i: (0, 0))
_DEGP = pl.BlockSpec((NW, RB), lambda i: (0, i))


def _splt(dh):
    return pl.BlockSpec((NC, RB, dh), lambda i: (0, i, 0))


def _outs(dh):
    return jax.ShapeDtypeStruct((NC, NP, dh), jnp.float32)


def _tc_first(x_pad, w, deg_parts):
    return pl.pallas_call(
        _tc_first_body, grid=(GRID,),
        in_specs=[_ROWS, _WMAT, _DEGP],
        out_specs=_splt(DH), out_shape=_outs(DH),
    )(x_pad, w, deg_parts)


def _tc_mid(s, deg_parts, w, cb, dh_out):
    return pl.pallas_call(
        functools.partial(_tc_mid_body, dh_out), grid=(GRID,),
        in_specs=[_splt(DH), _DEGP,
                  pl.BlockSpec((D, 2 * dh_out), lambda i: (0, 0)),
                  pl.BlockSpec((2, D), lambda i: (0, 0))],
        out_specs=_splt(dh_out), out_shape=_outs(dh_out),
    )(s, deg_parts, w, cb)


def _tc_out(s, deg_parts, b):
    return pl.pallas_call(
        _tc_out_body, grid=(GRID,),
        in_specs=[_splt(DH3), _DEGP,
                  pl.BlockSpec((1, 2 * DH3), lambda i: (0, 0))],
        out_specs=pl.BlockSpec((RB, 2 * DH3), lambda i: (i, 0)),
        out_shape=jax.ShapeDtypeStruct((NP, 2 * DH3), jnp.float32),
    )(s, deg_parts, b)


@jax.jit
def kernel(x, edge_index, W1, b1, W2, b2, W3, b3, g1, beta1, g2, beta2):
    src = edge_index[0]
    dst = edge_index[1]
    pad = EP - E
    src_pad = jnp.concatenate([src, jnp.zeros((pad,), jnp.int32)])
    dst_pad = jnp.concatenate([dst, jnp.full((pad,), N, jnp.int32)])
    src2d = src_pad.reshape(EP // CHUNK, CHUNK)
    dst2d = dst_pad.reshape(EP // CHUNK, CHUNK)
    idx2d = jnp.stack([src2d, dst2d], axis=1)
    x_pad = jnp.pad(x, ((0, NP - N), (0, 0)))

    bn = 1.0 / jnp.sqrt(jnp.float32(1.0 + 1e-5))
    c1 = g1 * bn
    cb1 = jnp.stack([c1, c1 * b1 + beta1])
    c2 = g2 * bn
    cb2 = jnp.stack([c2, c2 * b2 + beta2])
    w3_pad = jnp.pad(W3, ((0, 0), (0, 2 * DH3 - W3.shape[1])))
    b3_pad = jnp.pad(b3, (0, 2 * DH3 - b3.shape[0]))[None, :]

    deg_parts = _degree_histogram(dst_pad)

    h1 = _tc_first(x_pad, W1, deg_parts)
    s1 = _scatter_rows(h1, idx2d, DH)
    h2 = _tc_mid(s1, deg_parts, W2, cb1, DH)
    s2 = _scatter_rows(h2, idx2d, DH)
    h3 = _tc_mid(s2, deg_parts, w3_pad, cb2, DH3)
    s3 = _scatter_rows(h3, idx2d, DH3)
    out = _tc_out(s3, deg_parts, b3_pad)
    return out[:N, :W3.shape[1]]


# speedup vs baseline: 1.8776x; 1.0264x over previous
import functools

import jax
import jax.numpy as jnp
from jax import lax
from jax.experimental import pallas as pl
from jax.experimental.pallas import tpu as pltpu
from jax.experimental.pallas import tpu_sc as plsc

N = 10000
D = 128
E = 320000

NC = 2
NS = 16
NW = NC * NS
DH = D // NC

NP = 10240
RPT = NP // NS
EP = 327680
EPW = EP // NW
EPS = EP // NS
CHUNK = 128
NCH = EPS // CHUNK

DH3 = 32

RB = 1024
GRID = NP // RB

NBUF = 5


def _mesh():
    return plsc.VectorSubcoreMesh(core_axis_name="c", subcore_axis_name="s")


_SC_PARAMS = pltpu.CompilerParams(needs_layout_passes=False)
_SC_PARAMS_LINEAR = pltpu.CompilerParams(
    needs_layout_passes=False, use_tc_tiling_on_sc=False)


def _deg_body(dst_hbm, out_hbm, dst_v, hist_v):
    c = lax.axis_index("c")
    s = lax.axis_index("s")
    wid = s * NC + c
    pltpu.sync_copy(dst_hbm.at[pl.ds(wid * EPW, EPW)], dst_v)

    zeros16 = jnp.zeros((16,), jnp.float32)

    def zero_body(i, carry):
        hist_v[pl.ds(i * 16, 16)] = zeros16
        return carry

    lax.fori_loop(0, NP // 16, zero_body, 0)

    ones16 = jnp.ones((16,), jnp.float32)

    def body(i, carry):
        idx = dst_v[pl.ds(i * 16, 16)]
        plsc.addupdate_scatter(hist_v, [idx], ones16)
        return carry

    lax.fori_loop(0, EPW // 16, body, 0)
    pltpu.sync_copy(hist_v, out_hbm.at[wid])


def _degree_histogram(dst_pad):
    k = pl.kernel(
        _deg_body,
        out_type=jax.ShapeDtypeStruct((NW, NP), jnp.float32),
        mesh=_mesh(),
        compiler_params=_SC_PARAMS,
        scratch_types=[
            pltpu.VMEM((EPW,), jnp.int32),
            pltpu.VMEM((NP,), jnp.float32),
        ],
    )
    return k(dst_pad)


def _spmm_common(h_hbm, idx_hbm, y_sh, h_sh, ibufs, rows,
                 isems, gsems, ssems, psem, qsem, c, s):
    ibase = s * NCH

    def issue_i(ci, b):
        pltpu.async_copy(idx_hbm.at[ibase + ci], ibufs[b], isems[b])

    def wait_i(ci, b):
        pltpu.make_async_copy(idx_hbm.at[ibase + ci], ibufs[b],
                              isems[b]).wait()

    def issue_g(ci, b):
        pltpu.async_copy(h_sh.at[ibufs[b].at[0]], rows[b], gsems[b])

    def wait_g(ci, b):
        pltpu.make_async_copy(h_sh.at[ibufs[b].at[0]], rows[b],
                              gsems[b]).wait()

    def issue_s(ci, b):
        pltpu.async_copy(rows[b], y_sh.at[ibufs[b].at[1]], ssems[b],
                         add=True)

    def wait_s(ci, b):
        pltpu.make_async_copy(rows[b], y_sh.at[ibufs[b].at[1]],
                              ssems[b]).wait()

    def visit(ci, b, f_wait_s, f_issue_i, f_issue_g):
        bs = (b + 2) % NBUF
        bg = (b + 1) % NBUF
        if f_wait_s:
            wait_s(ci - 3, bs)
        if f_issue_i:
            issue_i(ci + 2, bs)
        if f_issue_g:
            wait_i(ci + 1, bg)
            issue_g(ci + 1, bg)
        wait_g(ci, b)
        issue_s(ci, b)

    pltpu.async_copy(h_hbm.at[c, pl.ds(s * RPT, RPT)],
                     h_sh.at[pl.ds(s * RPT, RPT)], psem)
    pltpu.async_copy(h_hbm.at[c, pl.ds(s * RPT, RPT)],
                     y_sh.at[pl.ds(s * RPT, RPT)], qsem)
    issue_i(0, 0)
    issue_i(1, 1)
    pltpu.make_async_copy(h_hbm.at[c, pl.ds(s * RPT, RPT)],
                          h_sh.at[pl.ds(s * RPT, RPT)], psem).wait()
    pltpu.make_async_copy(h_hbm.at[c, pl.ds(s * RPT, RPT)],
                          y_sh.at[pl.ds(s * RPT, RPT)], qsem).wait()
    plsc.subcore_barrier()
    wait_i(0, 0)
    issue_g(0, 0)

    for k in range(NBUF):
        visit(k, k, k >= 3, True, True)

    def round_body(o, carry):
        c0 = o * NBUF
        for k in range(NBUF):
            visit(c0 + k, k, True, True, True)
        return carry

    lax.fori_loop(1, NCH // NBUF - 1, round_body, 0)

    cl = NCH - NBUF
    for k in range(NBUF):
        visit(cl + k, k, True, cl + k + 2 < NCH, cl + k + 1 < NCH)
    for ci in range(NCH - 3, NCH):
        wait_s(ci, ci % NBUF)

    plsc.subcore_barrier()


def _spmm_body(h_hbm, idx_hbm, out_hbm,
               y_sh, h_sh, ibufs, rows, isems, gsems, ssems, psem, qsem):
    c = lax.axis_index("c")
    s = lax.axis_index("s")
    _spmm_common(h_hbm, idx_hbm, y_sh, h_sh, ibufs, rows,
                 isems, gsems, ssems, psem, qsem, c, s)
    pltpu.sync_copy(y_sh.at[pl.ds(s * RPT, RPT)],
                    out_hbm.at[c, pl.ds(s * RPT, RPT)])


def _spmm_out_body(h_hbm, idx_hbm, dinv_hbm, b_hbm, out_hbm,
                   y_sh, h_sh, ibufs, rows, isems, gsems, ssems,
                   psem, qsem, dv, bv):
    c = lax.axis_index("c")
    s = lax.axis_index("s")
    _spmm_common(h_hbm, idx_hbm, y_sh, h_sh, ibufs, rows,
                 isems, gsems, ssems, psem, qsem, c, s)
    pltpu.sync_copy(dinv_hbm.at[pl.ds(s * RPT, RPT)], dv)
    pltpu.sync_copy(b_hbm.at[c], bv)
    b0 = bv[pl.ds(0, 16)]
    b1 = bv[pl.ds(16, 16)]
    for k in range(RPT // CHUNK):
        pltpu.sync_copy(y_sh.at[pl.ds(s * RPT + k * CHUNK, CHUNK)], rows[k])

        def gbody(g, carry, k=k):
            d16 = dv[pl.ds(k * CHUNK + g * 16, 16)]
            for j in range(16):
                r = g * 16 + j
                d = d16[j]
                rows[k][r, pl.ds(0, 16)] = rows[k][r, pl.ds(0, 16)] * d + b0
                rows[k][r, pl.ds(16, 16)] = (rows[k][r, pl.ds(16, 16)] * d
                                             + b1)
            return carry

        lax.fori_loop(0, CHUNK // 16, gbody, 0)
        pltpu.sync_copy(rows[k],
                        out_hbm.at[pl.ds(s * RPT + k * CHUNK, CHUNK),
                                   pl.ds(c * DH3, DH3)])


def _scatter_rows(h_split, idx2d, dh):
    k = pl.kernel(
        _spmm_body,
        out_type=jax.ShapeDtypeStruct((NC, NP, dh), jnp.float32),
        mesh=_mesh(),
        compiler_params=_SC_PARAMS_LINEAR,
        scratch_types=[
            pltpu.VMEM_SHARED((NP, dh), jnp.float32),
            pltpu.VMEM_SHARED((NP, dh), jnp.float32),
            [pltpu.VMEM((2, CHUNK), jnp.int32) for _ in range(NBUF)],
            [pltpu.VMEM((CHUNK, dh), jnp.float32) for _ in range(NBUF)],
            [pltpu.SemaphoreType.DMA for _ in range(NBUF)],
            [pltpu.SemaphoreType.DMA for _ in range(NBUF)],
            [pltpu.SemaphoreType.DMA for _ in range(NBUF)],
            pltpu.SemaphoreType.DMA,
            pltpu.SemaphoreType.DMA,
        ],
    )
    return k(h_split, idx2d)


def _scatter_out(h_split, idx2d, dinv, b):
    k = pl.kernel(
        _spmm_out_body,
        out_type=jax.ShapeDtypeStruct((NP, 2 * DH3), jnp.float32),
        mesh=_mesh(),
        compiler_params=_SC_PARAMS_LINEAR,
        scratch_types=[
            pltpu.VMEM_SHARED((NP, DH3), jnp.float32),
            pltpu.VMEM_SHARED((NP, DH3), jnp.float32),
            [pltpu.VMEM((2, CHUNK), jnp.int32) for _ in range(NBUF)],
            [pltpu.VMEM((CHUNK, DH3), jnp.float32) for _ in range(NBUF)],
            [pltpu.SemaphoreType.DMA for _ in range(NBUF)],
            [pltpu.SemaphoreType.DMA for _ in range(NBUF)],
            [pltpu.SemaphoreType.DMA for _ in range(NBUF)],
            pltpu.SemaphoreType.DMA,
            pltpu.SemaphoreType.DMA,
            pltpu.VMEM((RPT,), jnp.float32),
            pltpu.VMEM((DH3,), jnp.float32),
        ],
    )
    return k(h_split, idx2d, dinv, b)


def _dinv_of(deg_ref):
    deg = jnp.sum(deg_ref[...], axis=0) + 1.0
    return lax.rsqrt(deg)


def _split(h, dh):
    return jnp.stack([h[:, :dh], h[:, dh:2 * dh]])


def _tc_first_body(x_ref, w_ref, deg_ref, o_ref):
    dinv = _dinv_of(deg_ref)
    h = jnp.dot(x_ref[...], w_ref[...], preferred_element_type=jnp.float32)
    o_ref[...] = _split(h * dinv[:, None], DH)


def _tc_mid_body(dh_out, s_ref, deg_ref, w_ref, cb_ref, o_ref):
    dinv = _dinv_of(deg_ref)
    sv = s_ref[...]
    z = jnp.concatenate([sv[0], sv[1]], axis=1)
    z = z * dinv[:, None]
    cb = cb_ref[...]
    xn = jnp.maximum(z * cb[0][None, :] + cb[1][None, :], 0.0)
    h = jnp.dot(xn, w_ref[...], preferred_element_type=jnp.float32)
    o_ref[...] = _split(h * dinv[:, None], dh_out)


def _tc_mid3_body(s_ref, deg_ref, w_ref, cb_ref, o_ref, dv_ref):
    dinv = _dinv_of(deg_ref)
    sv = s_ref[...]
    z = jnp.concatenate([sv[0], sv[1]], axis=1)
    z = z * dinv[:, None]
    cb = cb_ref[...]
    xn = jnp.maximum(z * cb[0][None, :] + cb[1][None, :], 0.0)
    h = jnp.dot(xn, w_ref[...], preferred_element_type=jnp.float32)
    o_ref[...] = _split(h * dinv[:, None], DH3)
    dv_ref[...] = dinv


_ROWS = pl.BlockSpec((RB, D), lambda i: (i, 0))
_WMAT = pl.BlockSpec((D, D), lambda i: (0, 0))
_DEGP = pl.BlockSpec((NW, RB), lambda i: (0, i))


def _splt(dh):
    return pl.BlockSpec((NC, RB, dh), lambda i: (0, i, 0))


def _outs(dh):
    return jax.ShapeDtypeStruct((NC, NP, dh), jnp.float32)


def _tc_first(x_pad, w, deg_parts):
    return pl.pallas_call(
        _tc_first_body, grid=(GRID,),
        in_specs=[_ROWS, _WMAT, _DEGP],
        out_specs=_splt(DH), out_shape=_outs(DH),
    )(x_pad, w, deg_parts)


def _tc_mid(s, deg_parts, w, cb, dh_out):
    return pl.pallas_call(
        functools.partial(_tc_mid_body, dh_out), grid=(GRID,),
        in_specs=[_splt(DH), _DEGP,
                  pl.BlockSpec((D, 2 * dh_out), lambda i: (0, 0)),
                  pl.BlockSpec((2, D), lambda i: (0, 0))],
        out_specs=_splt(dh_out), out_shape=_outs(dh_out),
    )(s, deg_parts, w, cb)


def _tc_mid3(s, deg_parts, w, cb):
    return pl.pallas_call(
        _tc_mid3_body, grid=(GRID,),
        in_specs=[_splt(DH), _DEGP,
                  pl.BlockSpec((D, 2 * DH3), lambda i: (0, 0)),
                  pl.BlockSpec((2, D), lambda i: (0, 0))],
        out_specs=[_splt(DH3), pl.BlockSpec((RB,), lambda i: (i,))],
        out_shape=[_outs(DH3),
                   jax.ShapeDtypeStruct((NP,), jnp.float32)],
    )(s, deg_parts, w, cb)


@jax.jit
def kernel(x, edge_index, W1, b1, W2, b2, W3, b3, g1, beta1, g2, beta2):
    src = edge_index[0]
    dst = edge_index[1]
    pad = EP - E
    src_pad = jnp.concatenate([src, jnp.zeros((pad,), jnp.int32)])
    dst_pad = jnp.concatenate([dst, jnp.full((pad,), N, jnp.int32)])
    src2d = src_pad.reshape(EP // CHUNK, CHUNK)
    dst2d = dst_pad.reshape(EP // CHUNK, CHUNK)
    idx2d = jnp.stack([src2d, dst2d], axis=1)
    x_pad = jnp.pad(x, ((0, NP - N), (0, 0)))

    bn = 1.0 / jnp.sqrt(jnp.float32(1.0 + 1e-5))
    c1 = g1 * bn
    cb1 = jnp.stack([c1, c1 * b1 + beta1])
    c2 = g2 * bn
    cb2 = jnp.stack([c2, c2 * b2 + beta2])
    w3_pad = jnp.pad(W3, ((0, 0), (0, 2 * DH3 - W3.shape[1])))
    b3_pad = jnp.pad(b3, (0, 2 * DH3 - b3.shape[0])).reshape(NC, DH3)

    deg_parts = _degree_histogram(dst_pad)

    h1 = _tc_first(x_pad, W1, deg_parts)
    s1 = _scatter_rows(h1, idx2d, DH)
    h2 = _tc_mid(s1, deg_parts, W2, cb1, DH)
    s2 = _scatter_rows(h2, idx2d, DH)
    h3, dinv3 = _tc_mid3(s2, deg_parts, w3_pad, cb2)
    out = _scatter_out(h3, idx2d, dinv3, b3_pad)
    return out[:N, :W3.shape[1]]


# speedup vs baseline: 1.9073x; 1.0158x over previous
import functools

import jax
import jax.numpy as jnp
from jax import lax
from jax.experimental import pallas as pl
from jax.experimental.pallas import tpu as pltpu
from jax.experimental.pallas import tpu_sc as plsc

N = 10000
D = 128
E = 320000

NC = 2
NS = 16
NW = NC * NS
DH = D // NC

NP = 10240
RPT = NP // NS
EP = 327680
EPW = EP // NW
EPS = EP // NS
CHUNK = 128
NCH = EPS // CHUNK

DH3 = 32

RB = 2048
GRID = NP // RB

NBUF = 5


def _mesh():
    return plsc.VectorSubcoreMesh(core_axis_name="c", subcore_axis_name="s")


_SC_PARAMS = pltpu.CompilerParams(needs_layout_passes=False)
_SC_PARAMS_LINEAR = pltpu.CompilerParams(
    needs_layout_passes=False, use_tc_tiling_on_sc=False)


def _deg_body(dst_hbm, out_hbm, dst_v, hist_v):
    c = lax.axis_index("c")
    s = lax.axis_index("s")
    wid = s * NC + c
    pltpu.sync_copy(dst_hbm.at[pl.ds(wid * EPW, EPW)], dst_v)

    zeros16 = jnp.zeros((16,), jnp.float32)

    def zero_body(i, carry):
        hist_v[pl.ds(i * 16, 16)] = zeros16
        return carry

    lax.fori_loop(0, NP // 16, zero_body, 0)

    ones16 = jnp.ones((16,), jnp.float32)

    def body(i, carry):
        idx = dst_v[pl.ds(i * 16, 16)]
        plsc.addupdate_scatter(hist_v, [idx], ones16)
        return carry

    lax.fori_loop(0, EPW // 16, body, 0)
    pltpu.sync_copy(hist_v, out_hbm.at[wid])


def _degree_histogram(dst_pad):
    k = pl.kernel(
        _deg_body,
        out_type=jax.ShapeDtypeStruct((NW, NP), jnp.float32),
        mesh=_mesh(),
        compiler_params=_SC_PARAMS,
        scratch_types=[
            pltpu.VMEM((EPW,), jnp.int32),
            pltpu.VMEM((NP,), jnp.float32),
        ],
    )
    return k(dst_pad)


def _spmm_common(h_hbm, idx_hbm, y_sh, h_sh, ibufs, rows,
                 isems, gsems, ssems, psem, qsem, c, s):
    ibase = s * NCH

    def issue_i(ci, b):
        pltpu.async_copy(idx_hbm.at[ibase + ci], ibufs[b], isems[b])

    def wait_i(ci, b):
        pltpu.make_async_copy(idx_hbm.at[ibase + ci], ibufs[b],
                              isems[b]).wait()

    def issue_g(ci, b):
        pltpu.async_copy(h_sh.at[ibufs[b].at[0]], rows[b], gsems[b])

    def wait_g(ci, b):
        pltpu.make_async_copy(h_sh.at[ibufs[b].at[0]], rows[b],
                              gsems[b]).wait()

    def issue_s(ci, b):
        pltpu.async_copy(rows[b], y_sh.at[ibufs[b].at[1]], ssems[b],
                         add=True)

    def wait_s(ci, b):
        pltpu.make_async_copy(rows[b], y_sh.at[ibufs[b].at[1]],
                              ssems[b]).wait()

    def visit(ci, b, f_wait_s, f_issue_i, f_issue_g):
        bs = (b + 2) % NBUF
        bg = (b + 1) % NBUF
        if f_wait_s:
            wait_s(ci - 3, bs)
        if f_issue_i:
            issue_i(ci + 2, bs)
        if f_issue_g:
            wait_i(ci + 1, bg)
            issue_g(ci + 1, bg)
        wait_g(ci, b)
        issue_s(ci, b)

    pltpu.async_copy(h_hbm.at[c, pl.ds(s * RPT, RPT)],
                     h_sh.at[pl.ds(s * RPT, RPT)], psem)
    pltpu.async_copy(h_hbm.at[c, pl.ds(s * RPT, RPT)],
                     y_sh.at[pl.ds(s * RPT, RPT)], qsem)
    issue_i(0, 0)
    issue_i(1, 1)
    pltpu.make_async_copy(h_hbm.at[c, pl.ds(s * RPT, RPT)],
                          h_sh.at[pl.ds(s * RPT, RPT)], psem).wait()
    pltpu.make_async_copy(h_hbm.at[c, pl.ds(s * RPT, RPT)],
                          y_sh.at[pl.ds(s * RPT, RPT)], qsem).wait()
    plsc.subcore_barrier()
    wait_i(0, 0)
    issue_g(0, 0)

    for k in range(NBUF):
        visit(k, k, k >= 3, True, True)

    def round_body(o, carry):
        c0 = o * NBUF
        for k in range(NBUF):
            visit(c0 + k, k, True, True, True)
        return carry

    lax.fori_loop(1, NCH // NBUF - 1, round_body, 0)

    cl = NCH - NBUF
    for k in range(NBUF):
        visit(cl + k, k, True, cl + k + 2 < NCH, cl + k + 1 < NCH)
    for ci in range(NCH - 3, NCH):
        wait_s(ci, ci % NBUF)

    plsc.subcore_barrier()


def _spmm_body(h_hbm, idx_hbm, out_hbm,
               y_sh, h_sh, ibufs, rows, isems, gsems, ssems, psem, qsem):
    c = lax.axis_index("c")
    s = lax.axis_index("s")
    _spmm_common(h_hbm, idx_hbm, y_sh, h_sh, ibufs, rows,
                 isems, gsems, ssems, psem, qsem, c, s)
    pltpu.sync_copy(y_sh.at[pl.ds(s * RPT, RPT)],
                    out_hbm.at[c, pl.ds(s * RPT, RPT)])


def _spmm_out_body(h_hbm, idx_hbm, dinv_hbm, b_hbm, out_hbm,
                   y_sh, h_sh, ibufs, rows, isems, gsems, ssems,
                   psem, qsem, dv, bv):
    c = lax.axis_index("c")
    s = lax.axis_index("s")
    _spmm_common(h_hbm, idx_hbm, y_sh, h_sh, ibufs, rows,
                 isems, gsems, ssems, psem, qsem, c, s)
    pltpu.sync_copy(dinv_hbm.at[pl.ds(s * RPT, RPT)], dv)
    pltpu.sync_copy(b_hbm.at[c], bv)
    b0 = bv[pl.ds(0, 16)]
    b1 = bv[pl.ds(16, 16)]
    for k in range(RPT // CHUNK):
        pltpu.sync_copy(y_sh.at[pl.ds(s * RPT + k * CHUNK, CHUNK)], rows[k])

        def gbody(g, carry, k=k):
            d16 = dv[pl.ds(k * CHUNK + g * 16, 16)]
            for j in range(16):
                r = g * 16 + j
                d = d16[j]
                rows[k][r, pl.ds(0, 16)] = rows[k][r, pl.ds(0, 16)] * d + b0
                rows[k][r, pl.ds(16, 16)] = (rows[k][r, pl.ds(16, 16)] * d
                                             + b1)
            return carry

        lax.fori_loop(0, CHUNK // 16, gbody, 0)
        pltpu.sync_copy(rows[k],
                        out_hbm.at[pl.ds(s * RPT + k * CHUNK, CHUNK),
                                   pl.ds(c * DH3, DH3)])


def _scatter_rows(h_split, idx2d, dh):
    k = pl.kernel(
        _spmm_body,
        out_type=jax.ShapeDtypeStruct((NC, NP, dh), jnp.float32),
        mesh=_mesh(),
        compiler_params=_SC_PARAMS_LINEAR,
        scratch_types=[
            pltpu.VMEM_SHARED((NP, dh), jnp.float32),
            pltpu.VMEM_SHARED((NP, dh), jnp.float32),
            [pltpu.VMEM((2, CHUNK), jnp.int32) for _ in range(NBUF)],
            [pltpu.VMEM((CHUNK, dh), jnp.float32) for _ in range(NBUF)],
            [pltpu.SemaphoreType.DMA for _ in range(NBUF)],
            [pltpu.SemaphoreType.DMA for _ in range(NBUF)],
            [pltpu.SemaphoreType.DMA for _ in range(NBUF)],
            pltpu.SemaphoreType.DMA,
            pltpu.SemaphoreType.DMA,
        ],
    )
    return k(h_split, idx2d)


def _scatter_out(h_split, idx2d, dinv, b):
    k = pl.kernel(
        _spmm_out_body,
        out_type=jax.ShapeDtypeStruct((NP, 2 * DH3), jnp.float32),
        mesh=_mesh(),
        compiler_params=_SC_PARAMS_LINEAR,
        scratch_types=[
            pltpu.VMEM_SHARED((NP, DH3), jnp.float32),
            pltpu.VMEM_SHARED((NP, DH3), jnp.float32),
            [pltpu.VMEM((2, CHUNK), jnp.int32) for _ in range(NBUF)],
            [pltpu.VMEM((CHUNK, DH3), jnp.float32) for _ in range(NBUF)],
            [pltpu.SemaphoreType.DMA for _ in range(NBUF)],
            [pltpu.SemaphoreType.DMA for _ in range(NBUF)],
            [pltpu.SemaphoreType.DMA for _ in range(NBUF)],
            pltpu.SemaphoreType.DMA,
            pltpu.SemaphoreType.DMA,
            pltpu.VMEM((RPT,), jnp.float32),
            pltpu.VMEM((DH3,), jnp.float32),
        ],
    )
    return k(h_split, idx2d, dinv, b)


def _dinv_of(deg_ref):
    deg = jnp.sum(deg_ref[...], axis=0) + 1.0
    return lax.rsqrt(deg)


def _split(h, dh):
    return jnp.stack([h[:, :dh], h[:, dh:2 * dh]])


def _tc_first_body(x_ref, w_ref, deg_ref, o_ref):
    dinv = _dinv_of(deg_ref)
    h = jnp.dot(x_ref[...], w_ref[...], preferred_element_type=jnp.float32)
    o_ref[...] = _split(h * dinv[:, None], DH)


def _tc_mid_body(dh_out, s_ref, deg_ref, w_ref, cb_ref, o_ref):
    dinv = _dinv_of(deg_ref)
    sv = s_ref[...]
    z = jnp.concatenate([sv[0], sv[1]], axis=1)
    z = z * dinv[:, None]
    cb = cb_ref[...]
    xn = jnp.maximum(z * cb[0][None, :] + cb[1][None, :], 0.0)
    h = jnp.dot(xn, w_ref[...], preferred_element_type=jnp.float32)
    o_ref[...] = _split(h * dinv[:, None], dh_out)


def _tc_mid3_body(s_ref, deg_ref, w_ref, cb_ref, o_ref, dv_ref):
    dinv = _dinv_of(deg_ref)
    sv = s_ref[...]
    z = jnp.concatenate([sv[0], sv[1]], axis=1)
    z = z * dinv[:, None]
    cb = cb_ref[...]
    xn = jnp.maximum(z * cb[0][None, :] + cb[1][None, :], 0.0)
    h = jnp.dot(xn, w_ref[...], preferred_element_type=jnp.float32)
    o_ref[...] = _split(h * dinv[:, None], DH3)
    dv_ref[...] = dinv


_ROWS = pl.BlockSpec((RB, D), lambda i: (i, 0))
_WMAT = pl.BlockSpec((D, D), lambda i: (0, 0))
_DEGP = pl.BlockSpec((NW, RB), lambda i: (0, i))


def _splt(dh):
    return pl.BlockSpec((NC, RB, dh), lambda i: (0, i, 0))


def _outs(dh):
    return jax.ShapeDtypeStruct((NC, NP, dh), jnp.float32)


def _tc_first(x_pad, w, deg_parts):
    return pl.pallas_call(
        _tc_first_body, grid=(GRID,),
        in_specs=[_ROWS, _WMAT, _DEGP],
        out_specs=_splt(DH), out_shape=_outs(DH),
    )(x_pad, w, deg_parts)


def _tc_mid(s, deg_parts, w, cb, dh_out):
    return pl.pallas_call(
        functools.partial(_tc_mid_body, dh_out), grid=(GRID,),
        in_specs=[_splt(DH), _DEGP,
                  pl.BlockSpec((D, 2 * dh_out), lambda i: (0, 0)),
                  pl.BlockSpec((2, D), lambda i: (0, 0))],
        out_specs=_splt(dh_out), out_shape=_outs(dh_out),
    )(s, deg_parts, w, cb)


def _tc_mid3(s, deg_parts, w, cb):
    return pl.pallas_call(
        _tc_mid3_body, grid=(GRID,),
        in_specs=[_splt(DH), _DEGP,
                  pl.BlockSpec((D, 2 * DH3), lambda i: (0, 0)),
                  pl.BlockSpec((2, D), lambda i: (0, 0))],
        out_specs=[_splt(DH3), pl.BlockSpec((RB,), lambda i: (i,))],
        out_shape=[_outs(DH3),
                   jax.ShapeDtypeStruct((NP,), jnp.float32)],
    )(s, deg_parts, w, cb)


@jax.jit
def kernel(x, edge_index, W1, b1, W2, b2, W3, b3, g1, beta1, g2, beta2):
    src = edge_index[0]
    dst = edge_index[1]
    pad = EP - E
    src_pad = jnp.concatenate([src, jnp.zeros((pad,), jnp.int32)])
    dst_pad = jnp.concatenate([dst, jnp.full((pad,), N, jnp.int32)])
    src2d = src_pad.reshape(EP // CHUNK, CHUNK)
    dst2d = dst_pad.reshape(EP // CHUNK, CHUNK)
    idx2d = jnp.stack([src2d, dst2d], axis=1)
    x_pad = jnp.pad(x, ((0, NP - N), (0, 0)))

    bn = 1.0 / jnp.sqrt(jnp.float32(1.0 + 1e-5))
    c1 = g1 * bn
    cb1 = jnp.stack([c1, c1 * b1 + beta1])
    c2 = g2 * bn
    cb2 = jnp.stack([c2, c2 * b2 + beta2])
    w3_pad = jnp.pad(W3, ((0, 0), (0, 2 * DH3 - W3.shape[1])))
    b3_pad = jnp.pad(b3, (0, 2 * DH3 - b3.shape[0])).reshape(NC, DH3)

    deg_parts = _degree_histogram(dst_pad)

    h1 = _tc_first(x_pad, W1, deg_parts)
    s1 = _scatter_rows(h1, idx2d, DH)
    h2 = _tc_mid(s1, deg_parts, W2, cb1, DH)
    s2 = _scatter_rows(h2, idx2d, DH)
    h3, dinv3 = _tc_mid3(s2, deg_parts, w3_pad, cb2)
    out = _scatter_out(h3, idx2d, dinv3, b3_pad)
    return out[:N, :W3.shape[1]]
